# baseline (speedup 1.0000x reference)
import functools

import jax
import jax.numpy as jnp
from jax import lax
from jax.experimental import pallas as pl
from jax.experimental.pallas import tpu as pltpu
from jax.experimental.pallas import tpu_sc as plsc

_NUM_CORES = 2
_NUM_SUBCORES = 16
_NUM_WORKERS = _NUM_CORES * _NUM_SUBCORES
_LANES = 16


def _make_sc_kernel(n_nodes, n_subtok, emb_dim, nodes_per_chunk):
  assert n_nodes % _NUM_WORKERS == 0
  nodes_per_worker = n_nodes // _NUM_WORKERS
  assert nodes_per_worker % nodes_per_chunk == 0
  num_chunks = nodes_per_worker // nodes_per_chunk
  assert num_chunks % 4 == 0
  assert emb_dim % _LANES == 0
  groups = emb_dim // _LANES
  scale = 1.0 / float(n_nodes)

  mesh = plsc.VectorSubcoreMesh(
      core_axis_name="c", subcore_axis_name="s",
      num_cores=_NUM_CORES, num_subcores=_NUM_SUBCORES)

  @functools.partial(
      pl.kernel,
      out_type=jax.ShapeDtypeStruct((n_nodes, emb_dim), jnp.float32),
      mesh=mesh,
      scratch_types=[
          pltpu.VMEM((n_subtok, nodes_per_chunk), jnp.int32),
          pltpu.VMEM((n_subtok, nodes_per_chunk), jnp.int32),
          pltpu.VMEM((n_subtok, nodes_per_chunk), jnp.int32),
          pltpu.VMEM((n_subtok, nodes_per_chunk), jnp.int32),
          pltpu.VMEM((n_subtok * nodes_per_chunk, emb_dim), jnp.bfloat16),
          pltpu.VMEM((n_subtok * nodes_per_chunk, emb_dim), jnp.bfloat16),
          pltpu.VMEM((nodes_per_chunk, emb_dim), jnp.float32),
          pltpu.SemaphoreType.DMA,
          pltpu.SemaphoreType.DMA,
          pltpu.SemaphoreType.DMA,
          pltpu.SemaphoreType.DMA,
          pltpu.SemaphoreType.DMA,
          pltpu.SemaphoreType.DMA,
      ],
      compiler_params=pltpu.CompilerParams(
          use_tc_tiling_on_sc=False, needs_layout_passes=False),
  )
  def sc_kernel(x_hbm, table_hbm, out_hbm,
                idx0, idx1, idx2, idx3, rows0, rows1, out_v,
                isem0, isem1, isem2, isem3, gsem0, gsem1):
    wid = lax.axis_index("s") * _NUM_CORES + lax.axis_index("c")
    node_base = wid * nodes_per_worker

    def chunk_slice(chunk):
      return pl.ds(node_base + chunk * nodes_per_chunk, nodes_per_chunk)

    def idx_fire(chunk, idx_v, isem):
      pltpu.async_copy(x_hbm.at[:, chunk_slice(chunk)], idx_v, isem)

    def idx_wait(idx_v, isem):
      pltpu.make_async_copy(
          x_hbm.at[:, pl.ds(0, nodes_per_chunk)], idx_v, isem).wait()

    def gather_fire(idx_v, rows_v, sem):
      for j in range(n_subtok):
        pltpu.async_copy(table_hbm.at[idx_v.at[j, :]],
                         rows_v.at[pl.ds(j * nodes_per_chunk, nodes_per_chunk)],
                         sem)

    def gather_drain(idx_v, rows_v, sem):
      for j in range(n_subtok):
        pltpu.make_async_copy(
            table_hbm.at[idx_v.at[j, :]],
            rows_v.at[pl.ds(j * nodes_per_chunk, nodes_per_chunk)],
            sem).wait()

    iota = lax.broadcasted_iota(jnp.int32, (_LANES,), 0)
    lane0 = iota * 0
    halves = groups // 2
    cols = [[iota * 2 + h * 2 * _LANES + p for p in (0, 1)]
            for h in range(halves)]

    def compute_store(chunk, rows_v):
      def node_body(n, _):
        rown = lane0 + n
        for h in range(halves):
          dsl = pl.ds(h * 2 * _LANES, 2 * _LANES)
          acc_a, acc_b = plsc.unpack(
              rows_v[n, dsl], format=plsc.PackFormat.INTERLEAVED,
              preferred_element_type=jnp.float32)
          for j in range(1, n_subtok):
            a, b = plsc.unpack(
                rows_v[j * nodes_per_chunk + n, dsl],
                format=plsc.PackFormat.INTERLEAVED,
                preferred_element_type=jnp.float32)
            acc_a = acc_a + a
            acc_b = acc_b + b
          plsc.store_scatter(out_v, [rown, cols[h][0]], acc_a * scale)
          plsc.store_scatter(out_v, [rown, cols[h][1]], acc_b * scale)
        return 0

      lax.fori_loop(0, nodes_per_chunk, node_body, 0)
      pltpu.sync_copy(out_v, out_hbm.at[chunk_slice(chunk), :])

    idx_fire(0, idx0, isem0)
    idx_fire(1, idx1, isem1)
    idx_wait(idx0, isem0)
    gather_fire(idx0, rows0, gsem0)

    def body4(i, _):
      c = 4 * i
      idx_wait(idx1, isem1)
      gather_fire(idx1, rows1, gsem1)
      idx_fire(c + 2, idx2, isem2)
      gather_drain(idx0, rows0, gsem0)
      compute_store(c, rows0)

      idx_wait(idx2, isem2)
      gather_fire(idx2, rows0, gsem0)
      idx_fire(c + 3, idx3, isem3)
      gather_drain(idx1, rows1, gsem1)
      compute_store(c + 1, rows1)

      idx_wait(idx3, isem3)
      gather_fire(idx3, rows1, gsem1)

      @pl.when(c + 4 < num_chunks)
      def _():
        idx_fire(c + 4, idx0, isem0)

      gather_drain(idx2, rows0, gsem0)
      compute_store(c + 2, rows0)

      @pl.when(c + 4 < num_chunks)
      def _():
        idx_wait(idx0, isem0)
        gather_fire(idx0, rows0, gsem0)

      @pl.when(c + 5 < num_chunks)
      def _():
        idx_fire(c + 5, idx1, isem1)

      gather_drain(idx3, rows1, gsem1)
      compute_store(c + 3, rows1)
      return 0

    lax.fori_loop(0, num_chunks // 4, body4, 0)

  return sc_kernel


def kernel(x, emb_table):
  n_nodes, n_subtok = x.shape
  _, emb_dim = emb_table.shape
  sc = _make_sc_kernel(n_nodes, n_subtok, emb_dim, nodes_per_chunk=32)
  return sc(x.T.astype(jnp.int32), emb_table.astype(jnp.bfloat16))


# speedup vs baseline: 1.2320x; 1.2320x over previous
import functools

import jax
import jax.numpy as jnp
from jax import lax
from jax.experimental import pallas as pl
from jax.experimental.pallas import tpu as pltpu
from jax.experimental.pallas import tpu_sc as plsc

_NUM_CORES = 2
_NUM_SUBCORES = 16
_NUM_WORKERS = _NUM_CORES * _NUM_SUBCORES
_LANES = 16


def _make_sc_kernel(n_nodes, n_subtok, emb_dim, nodes_per_chunk):
  assert n_nodes % _NUM_WORKERS == 0
  nodes_per_worker = n_nodes // _NUM_WORKERS
  assert nodes_per_worker % nodes_per_chunk == 0
  num_chunks = nodes_per_worker // nodes_per_chunk
  assert num_chunks % 4 == 0
  assert emb_dim % _LANES == 0
  groups = emb_dim // _LANES
  scale = 1.0 / float(n_nodes)

  mesh = plsc.VectorSubcoreMesh(
      core_axis_name="c", subcore_axis_name="s",
      num_cores=_NUM_CORES, num_subcores=_NUM_SUBCORES)

  @functools.partial(
      pl.kernel,
      out_type=jax.ShapeDtypeStruct((n_nodes, emb_dim), jnp.float32),
      mesh=mesh,
      scratch_types=[
          pltpu.VMEM((n_subtok, nodes_per_chunk), jnp.int32),
          pltpu.VMEM((n_subtok, nodes_per_chunk), jnp.int32),
          pltpu.VMEM((n_subtok, nodes_per_chunk), jnp.int32),
          pltpu.VMEM((n_subtok, nodes_per_chunk), jnp.int32),
          pltpu.VMEM((n_subtok * nodes_per_chunk, emb_dim), jnp.float32),
          pltpu.VMEM((n_subtok * nodes_per_chunk, emb_dim), jnp.float32),
          pltpu.VMEM((nodes_per_chunk, emb_dim), jnp.float32),
          pltpu.SemaphoreType.DMA,
          pltpu.SemaphoreType.DMA,
          pltpu.SemaphoreType.DMA,
          pltpu.SemaphoreType.DMA,
          pltpu.SemaphoreType.DMA,
          pltpu.SemaphoreType.DMA,
      ],
      compiler_params=pltpu.CompilerParams(
          use_tc_tiling_on_sc=False, needs_layout_passes=False),
  )
  def sc_kernel(x_hbm, table_hbm, out_hbm,
                idx0, idx1, idx2, idx3, rows0, rows1, out_v,
                isem0, isem1, isem2, isem3, gsem0, gsem1):
    wid = lax.axis_index("s") * _NUM_CORES + lax.axis_index("c")
    node_base = wid * nodes_per_worker

    def chunk_slice(chunk):
      return pl.ds(node_base + chunk * nodes_per_chunk, nodes_per_chunk)

    def idx_fire(chunk, idx_v, isem):
      pltpu.async_copy(x_hbm.at[:, chunk_slice(chunk)], idx_v, isem)

    def idx_wait(idx_v, isem):
      pltpu.make_async_copy(
          x_hbm.at[:, pl.ds(0, nodes_per_chunk)], idx_v, isem).wait()

    def gather_fire(idx_v, rows_v, sem):
      for j in range(n_subtok):
        pltpu.async_copy(table_hbm.at[idx_v.at[j, :]],
                         rows_v.at[pl.ds(j * nodes_per_chunk, nodes_per_chunk)],
                         sem)

    def gather_drain(idx_v, rows_v, sem):
      for j in range(n_subtok):
        pltpu.make_async_copy(
            table_hbm.at[idx_v.at[j, :]],
            rows_v.at[pl.ds(j * nodes_per_chunk, nodes_per_chunk)],
            sem).wait()

    def compute_store(chunk, rows_v):
      def node_body(n, _):
        for g in range(groups):
          dsl = pl.ds(g * _LANES, _LANES)
          acc = rows_v[n, dsl]
          for j in range(1, n_subtok):
            acc = acc + rows_v[j * nodes_per_chunk + n, dsl]
          out_v[n, dsl] = acc * scale
        return 0

      lax.fori_loop(0, nodes_per_chunk, node_body, 0)
      pltpu.sync_copy(out_v, out_hbm.at[chunk_slice(chunk), :])

    idx_fire(0, idx0, isem0)
    idx_fire(1, idx1, isem1)
    idx_wait(idx0, isem0)
    gather_fire(idx0, rows0, gsem0)

    def body4(i, _):
      c = 4 * i
      idx_wait(idx1, isem1)
      gather_fire(idx1, rows1, gsem1)
      idx_fire(c + 2, idx2, isem2)
      gather_drain(idx0, rows0, gsem0)
      compute_store(c, rows0)

      idx_wait(idx2, isem2)
      gather_fire(idx2, rows0, gsem0)
      idx_fire(c + 3, idx3, isem3)
      gather_drain(idx1, rows1, gsem1)
      compute_store(c + 1, rows1)

      idx_wait(idx3, isem3)
      gather_fire(idx3, rows1, gsem1)

      @pl.when(c + 4 < num_chunks)
      def _():
        idx_fire(c + 4, idx0, isem0)

      gather_drain(idx2, rows0, gsem0)
      compute_store(c + 2, rows0)

      @pl.when(c + 4 < num_chunks)
      def _():
        idx_wait(idx0, isem0)
        gather_fire(idx0, rows0, gsem0)

      @pl.when(c + 5 < num_chunks)
      def _():
        idx_fire(c + 5, idx1, isem1)

      gather_drain(idx3, rows1, gsem1)
      compute_store(c + 3, rows1)
      return 0

    lax.fori_loop(0, num_chunks // 4, body4, 0)

  return sc_kernel


def kernel(x, emb_table):
  n_nodes, n_subtok = x.shape
  _, emb_dim = emb_table.shape
  sc = _make_sc_kernel(n_nodes, n_subtok, emb_dim, nodes_per_chunk=32)
  return sc(x.T.astype(jnp.int32), emb_table.astype(jnp.float32))
